# Initial kernel scaffold; baseline (speedup 1.0000x reference)
import jax
import jax.numpy as jnp
from jax.experimental import pallas as pl


def kernel(logits, temperatures, top_ps, top_ks, min_ps):
    raise NotImplementedError("write your pallas kernel here")


# speedup vs baseline: 50.3228x; 50.3228x over previous
import functools

import numpy as np
import jax
import jax.numpy as jnp
from jax import lax
from jax.experimental import pallas as pl

MIN_TEMPERATURE = np.float32(1e-8)
MIN_PROB = np.float32(1e-10)
LOG_MIN_PROB = np.float32(np.log(np.float32(1e-10)))
INT_MIN = np.int32(-2**31)
INT_MAX = np.int32(2**31 - 1)
ROW_BLOCK = 8


def _floor_avg(lo, hi):
    return (lo >> 1) + (hi >> 1) + (lo & hi & 1)


def _sampler_body(nbits_v, logits_ref, t_ref, tp_ref, tk_ref, mp_ref, g_ref,
                  out_ref):
    V = logits_ref.shape[1]
    rb = logits_ref.shape[0]
    t = t_ref[0, 0, :].reshape(rb, 1)
    top_p = tp_ref[0, 0, :].reshape(rb, 1)
    min_p = mp_ref[0, 0, :].reshape(rb, 1)
    k = tk_ref[0, 0, :].reshape(rb, 1)

    x = logits_ref[...] / t
    x = x + np.float32(0.0)
    ibits = jax.lax.bitcast_convert_type(x, jnp.int32)
    skey = ibits ^ ((ibits >> 31) & np.int32(0x7FFFFFFF))
    iota = lax.broadcasted_iota(jnp.int32, (rb, V), 1)

    m = jnp.max(x, axis=-1, keepdims=True)
    mkey = jnp.max(skey, axis=-1, keepdims=True)

    active = (k > 0) & (k < V)
    k_eff = jnp.where(active, k, V)

    def tk_body(_, lh):
        lo, hi = lh
        mid = _floor_avg(lo, hi)
        cnt = jnp.sum((skey > mid).astype(jnp.int32), axis=-1, keepdims=True)
        pred = cnt >= k_eff
        return jnp.where(pred, mid + 1, lo), jnp.where(pred, hi, mid)

    lo = jnp.full((rb, 1), INT_MIN, jnp.int32)
    hi = jnp.full((rb, 1), INT_MAX, jnp.int32)
    tk_key, _ = lax.fori_loop(0, 32, tk_body, (lo, hi))

    n_gt = jnp.sum((skey > tk_key).astype(jnp.int32), axis=-1, keepdims=True)
    r_k = k_eff - n_gt

    def jk_body(_, lh):
        lo, hi = lh
        mid = _floor_avg(lo, hi)
        cnt = jnp.sum(((skey == tk_key) & (iota <= mid)).astype(jnp.int32),
                      axis=-1, keepdims=True)
        pred = cnt >= r_k
        return jnp.where(pred, lo, mid + 1), jnp.where(pred, mid, hi)

    lo = jnp.zeros((rb, 1), jnp.int32)
    hi = jnp.full((rb, 1), V - 1, jnp.int32)
    j_k, _ = lax.fori_loop(0, nbits_v, jk_body, (lo, hi))

    kept1 = (skey > tk_key) | ((skey == tk_key) & (iota <= j_k))

    e = jnp.exp(x - m)
    z1 = jnp.sum(jnp.where(kept1, e, np.float32(0.0)), axis=-1, keepdims=True)
    p1 = jnp.where(kept1, e / z1, np.float32(0.0))

    minkey1 = jnp.min(jnp.where(kept1, skey, INT_MAX), axis=-1, keepdims=True)

    def tp_body(_, lh):
        lo, hi = lh
        mid = _floor_avg(lo, hi)
        d = jnp.sum(jnp.where(skey > mid, p1, np.float32(0.0)),
                    axis=-1, keepdims=True)
        pred = d <= top_p
        return jnp.where(pred, lo, mid + 1), jnp.where(pred, mid, hi)

    tp_key, _ = lax.fori_loop(0, 32, tp_body, (minkey1, mkey))

    d_t = jnp.sum(jnp.where(skey > tp_key, p1, np.float32(0.0)),
                  axis=-1, keepdims=True)
    eq_p = kept1 & (skey == tp_key)
    p1_t = jnp.max(jnp.where(eq_p, p1, np.float32(0.0)), axis=-1, keepdims=True)
    n_eq_p = jnp.sum(eq_p.astype(jnp.int32), axis=-1, keepdims=True)
    q = (top_p - d_t) / p1_t
    r_p = jnp.minimum(q, n_eq_p.astype(jnp.float32)).astype(jnp.int32) + 1
    r_p = jnp.minimum(r_p, n_eq_p)

    def jp_body(_, lh):
        lo, hi = lh
        mid = _floor_avg(lo, hi)
        cnt = jnp.sum((eq_p & (iota <= mid)).astype(jnp.int32),
                      axis=-1, keepdims=True)
        pred = cnt >= r_p
        return jnp.where(pred, lo, mid + 1), jnp.where(pred, mid, hi)

    lo = jnp.zeros((rb, 1), jnp.int32)
    hi = jnp.full((rb, 1), V - 1, jnp.int32)
    j_p, _ = lax.fori_loop(0, nbits_v, jp_body, (lo, hi))

    kept2 = kept1 & ((skey > tp_key) | ((skey == tp_key) & (iota <= j_p)))

    z2 = jnp.sum(jnp.where(kept2, e, np.float32(0.0)), axis=-1, keepdims=True)
    thr = min_p * (np.float32(1.0) / z2)
    kept3 = kept2 & jnp.logical_not((e / z2) < thr)

    z3 = jnp.sum(jnp.where(kept3, e, np.float32(0.0)), axis=-1, keepdims=True)
    lz3 = jnp.log(z3)
    lp = jnp.where(kept3, jnp.maximum(x - m - lz3, LOG_MIN_PROB), LOG_MIN_PROB)
    f = g_ref[...] + lp
    fmax = jnp.max(f, axis=-1, keepdims=True)
    tok = jnp.min(jnp.where(f == fmax, iota, V), axis=-1)
    out_ref[0, 0, :] = tok


def _sample(logits, t, top_ps, top_ks, min_ps, g):
    B, V = logits.shape
    rb = ROW_BLOCK
    nblk = B // rb
    nbits_v = max(1, int(np.ceil(np.log2(V))))

    def r3(a, dtype):
        return a.astype(dtype).reshape(nblk, 1, rb)

    grid = (nblk,)
    row_spec = pl.BlockSpec((rb, V), lambda i: (i, 0))
    s_spec = pl.BlockSpec((1, 1, rb), lambda i: (i, 0, 0))

    out = pl.pallas_call(
        functools.partial(_sampler_body, nbits_v),
        grid=grid,
        in_specs=[row_spec, s_spec, s_spec, s_spec, s_spec, row_spec],
        out_specs=s_spec,
        out_shape=jax.ShapeDtypeStruct((nblk, 1, rb), jnp.int32),
    )(logits, r3(t, jnp.float32), r3(top_ps, jnp.float32),
      r3(top_ks, jnp.int32), r3(min_ps, jnp.float32), g)
    return out.reshape(B)


def kernel(logits, temperatures, top_ps, top_ks, min_ps):
    B, V = logits.shape
    t = jnp.maximum(temperatures, MIN_TEMPERATURE)
    g = jax.random.gumbel(jax.random.key(123), (B, V), jnp.float32)
    return _sample(logits.astype(jnp.float32), t, top_ps, top_ks, min_ps, g)
